# baseline (speedup 1.0000x reference)
import functools

import jax
import jax.numpy as jnp
import numpy as np
from jax import lax
from jax.experimental import pallas as pl
from jax.experimental.pallas import tpu as pltpu
from jax.experimental.pallas import tpu_sc as plsc

N = 10000
NPAD = 10240
E = 320000
CHUNK = 128
NCHUNKS = 2560
CH_PER_TILE = NCHUNKS // 32
EPAD = NCHUNKS * CHUNK
HEADS = 8
BASES = 4
F_H = 16
F_B = BASES * F_H
ROWS_PER_TILE = NPAD // 16
BLK = 512
GRID = 20

_mesh = plsc.VectorSubcoreMesh(core_axis_name="c", subcore_axis_name="s")
_sc_params = pltpu.CompilerParams(use_tc_tiling_on_sc=False,
                                  needs_layout_passes=False)


@functools.partial(
    pl.kernel,
    out_type=jax.ShapeDtypeStruct((2, NPAD // 128, 128), jnp.float32),
    mesh=_mesh,
    compiler_params=_sc_params,
    scratch_types=[
        pltpu.VMEM((CH_PER_TILE, CHUNK), jnp.int32),
        pltpu.VMEM((CHUNK, 16), jnp.float32),
        pltpu.VMEM((ROWS_PER_TILE, 16), jnp.float32),
        pltpu.VMEM((ROWS_PER_TILE // 128, 128), jnp.float32),
        pltpu.VMEM_SHARED((NPAD, 16), jnp.float32),
        pltpu.SemaphoreType.DMA,
    ],
)
def _sc_degree(col_hbm, out_hbm, col_v, ones_v, deg_loc, compact_v, deg_sh, sem):
    cid = lax.axis_index("c")
    sid = lax.axis_index("s")
    wid = sid * 2 + cid

    z16 = jnp.zeros((16,), jnp.float32)

    @pl.loop(0, CHUNK)
    def _(i):
        ones_v[i, pl.ds(0, 16)] = z16

    @pl.loop(0, ROWS_PER_TILE // CHUNK)
    def _(k):
        pltpu.sync_copy(ones_v, deg_sh.at[pl.ds(sid * ROWS_PER_TILE + k * CHUNK, CHUNK)])

    o16 = jnp.ones((16,), jnp.float32)

    @pl.loop(0, CHUNK)
    def _(i):
        ones_v[i, pl.ds(0, 16)] = o16

    pltpu.sync_copy(col_hbm.at[pl.ds(wid * CH_PER_TILE, CH_PER_TILE)], col_v)
    plsc.subcore_barrier()

    @pl.loop(0, CH_PER_TILE)
    def _(j):
        pltpu.async_copy(ones_v, deg_sh.at[col_v.at[j]], sem, add=True)

    @pl.loop(0, CH_PER_TILE)
    def _(j):
        pltpu.make_async_copy(ones_v, deg_sh.at[col_v.at[j]], sem).wait()

    plsc.subcore_barrier()

    pltpu.sync_copy(deg_sh.at[pl.ds(sid * ROWS_PER_TILE, ROWS_PER_TILE)], deg_loc)
    ivec16 = lax.iota(jnp.int32, 16)
    zvec16 = jnp.zeros((16,), jnp.int32)

    @pl.loop(0, ROWS_PER_TILE // 128)
    def _(gg):
        for k in range(8):
            v = plsc.load_gather(deg_loc, [ivec16 + (gg * 128 + 16 * k), zvec16])
            compact_v[gg, pl.ds(k * 16, 16)] = v

    pltpu.sync_copy(
        compact_v,
        out_hbm.at[cid, pl.ds(sid * (ROWS_PER_TILE // 128), ROWS_PER_TILE // 128)],
    )


F_T = F_B // 16
ECH_ROWS = 16
ROWS_CORE = NCHUNKS // 2
NECH = ROWS_CORE // ECH_ROWS
NECH_A = 36
NECH_B = 2 * NECH - NECH_A


@functools.partial(
    pl.kernel,
    out_type=jax.ShapeDtypeStruct((2, F_B, NPAD), jnp.float32),
    mesh=_mesh,
    compiler_params=_sc_params,
    scratch_types=[
        pltpu.VMEM((F_T, NPAD), jnp.float32),
        pltpu.VMEM((F_T, NPAD), jnp.float32),
        pltpu.VMEM((2, ECH_ROWS, CHUNK), jnp.int32),
        pltpu.VMEM((2, ECH_ROWS, CHUNK), jnp.int32),
        pltpu.SemaphoreType.DMA,
        pltpu.SemaphoreType.DMA,
        pltpu.SemaphoreType.DMA,
    ],
)
def _sc_agg(b2_hbm, row_hbm, col_hbm, out_hbm, b2_v, agg_v, row_v, col_v,
            sem_r, sem_c, sem_b):
    cid = lax.axis_index("c")
    sid = lax.axis_index("s")

    b2cp = pltpu.async_copy(b2_hbm.at[pl.ds(F_T * sid, F_T)], b2_v, sem_b)

    z16 = jnp.zeros((16,), jnp.float32)
    for j in range(F_T):
        @pl.loop(0, NPAD, step=16)
        def _(i):
            agg_v[j, pl.ds(i, 16)] = z16

    nech_c = jnp.where(cid == 0, NECH_A, NECH_B)
    rbase = jnp.where(cid == 0, 0, NECH_A * ECH_ROWS)

    def _fetch(ch, slot):
        pltpu.async_copy(
            row_hbm.at[pl.ds(rbase + ch * ECH_ROWS, ECH_ROWS)], row_v.at[slot], sem_r)
        pltpu.async_copy(
            col_hbm.at[pl.ds(rbase + ch * ECH_ROWS, ECH_ROWS)], col_v.at[slot], sem_c)

    def _wait(ch, slot):
        pltpu.make_async_copy(
            row_hbm.at[pl.ds(rbase + ch * ECH_ROWS, ECH_ROWS)], row_v.at[slot], sem_r).wait()
        pltpu.make_async_copy(
            col_hbm.at[pl.ds(rbase + ch * ECH_ROWS, ECH_ROWS)], col_v.at[slot], sem_c).wait()

    jvecs = [jnp.full((16,), j, jnp.int32) for j in range(F_T)]

    def _process(slot):
        @plsc.parallel_loop(0, ECH_ROWS, unroll=4)
        def _(r):
            for k in range(CHUNK // 16):
                r16 = row_v[slot, r, pl.ds(k * 16, 16)]
                c16 = col_v[slot, r, pl.ds(k * 16, 16)]
                for j in range(F_T):
                    v = plsc.load_gather(b2_v, [jvecs[j], r16])
                    plsc.addupdate_scatter(agg_v, [jvecs[j], c16], v)

    _fetch(0, 0)
    b2cp.wait()

    @pl.loop(0, NECH_B, step=2)
    def _(ch):
        @pl.when(ch < nech_c)
        def _():
            _wait(ch, 0)
            _fetch(ch + 1, 1)
            _process(0)
            _wait(ch + 1, 1)

            @pl.when(ch + 2 < nech_c)
            def _():
                _fetch(ch + 2, 0)

            _process(1)

    pltpu.sync_copy(agg_v, out_hbm.at[cid, pl.ds(F_T * sid, F_T)])


def _dense_body(x_ref, wb_ref, wc_ref, bc_ref, wr_ref, br_ref, bt_ref, wt_ref, r_ref):
    xb = x_ref[...]
    bt_ref[...] = lax.dot_general(wb_ref[...], xb, (((0,), (1,)), ((), ())),
                                  preferred_element_type=jnp.float32)
    wt_ref[...] = jnp.dot(xb, wc_ref[...], preferred_element_type=jnp.float32) + bc_ref[...]
    r_ref[...] = jnp.dot(xb, wr_ref[...], preferred_element_type=jnp.float32) + br_ref[...]


_dense = pl.pallas_call(
    _dense_body,
    grid=(GRID,),
    in_specs=[
        pl.BlockSpec((BLK, 128), lambda i: (i, 0)),
        pl.BlockSpec((128, F_B), lambda i: (0, 0)),
        pl.BlockSpec((128, HEADS * BASES), lambda i: (0, 0)),
        pl.BlockSpec((1, HEADS * BASES), lambda i: (0, 0)),
        pl.BlockSpec((128, 128), lambda i: (0, 0)),
        pl.BlockSpec((1, 128), lambda i: (0, 0)),
    ],
    out_specs=[
        pl.BlockSpec((F_B, BLK), lambda i: (0, i)),
        pl.BlockSpec((BLK, HEADS * BASES), lambda i: (i, 0)),
        pl.BlockSpec((BLK, 128), lambda i: (i, 0)),
    ],
    out_shape=[
        jax.ShapeDtypeStruct((F_B, NPAD), jnp.float32),
        jax.ShapeDtypeStruct((N, HEADS * BASES), jnp.float32),
        jax.ShapeDtypeStruct((N, 128), jnp.float32),
    ],
)


SBLK = 1024


def _scale_body(dp0_ref, dp1_ref, bases_ref, b2_ref, dis_ref):
    d = dp0_ref[...][0] + dp1_ref[...][0]
    deg_row = jnp.concatenate([d[r:r + 1, :] for r in range(8)], axis=1) + 1.0
    dis_row = lax.rsqrt(deg_row)
    dis_ref[...] = dis_row
    b2_ref[...] = bases_ref[...] * dis_row


_scale = pl.pallas_call(
    _scale_body,
    grid=(NPAD // SBLK,),
    in_specs=[
        pl.BlockSpec((1, 8, 128), lambda i: (0, i, 0)),
        pl.BlockSpec((1, 8, 128), lambda i: (1, i, 0)),
        pl.BlockSpec((F_B, SBLK), lambda i: (0, i)),
    ],
    out_specs=[
        pl.BlockSpec((F_B, SBLK), lambda i: (0, i)),
        pl.BlockSpec((1, SBLK), lambda i: (0, i)),
    ],
    out_shape=[
        jax.ShapeDtypeStruct((F_B, NPAD), jnp.float32),
        jax.ShapeDtypeStruct((1, NPAD), jnp.float32),
    ],
)


_P_np = np.zeros((BASES, HEADS * BASES, 128), np.float32)
_Q_np = np.zeros((BASES, F_B, 128), np.float32)
for _b in range(BASES):
    for _h in range(HEADS):
        for _f in range(F_H):
            _P_np[_b, _h * BASES + _b, _h * F_H + _f] = 1.0
            _Q_np[_b, _b * F_H + _f, _h * F_H + _f] = 1.0


def _finish_body(a0_ref, a1_ref, dis_ref, bases_ref, wt_ref, res_ref, bc_ref,
                 g_ref, bt_ref, p_ref, q_ref, o_ref):
    disr = dis_ref[...]
    a_t = a0_ref[...][0] + a1_ref[...][0]
    aggf_t = disr * a_t + (disr * disr) * bases_ref[...]
    wt = wt_ref[...]
    conv = None
    for b in range(BASES):
        we = jnp.dot(wt, p_ref[b], preferred_element_type=jnp.float32)
        ae = lax.dot_general(aggf_t, q_ref[b], (((0,), (0,)), ((), ())),
                             preferred_element_type=jnp.float32)
        t = we * ae
        conv = t if conv is None else conv + t
    o = conv + bc_ref[...] + res_ref[...]
    mu = jnp.mean(o, axis=1, keepdims=True)
    var = jnp.mean((o - mu) * (o - mu), axis=1, keepdims=True)
    o = (o - mu) * lax.rsqrt(var + 1e-5) * g_ref[...] + bt_ref[...]
    o_ref[...] = jnp.maximum(o, 0.0)


_finish = pl.pallas_call(
    _finish_body,
    grid=(GRID,),
    in_specs=[
        pl.BlockSpec((1, F_B, BLK), lambda i: (0, 0, i)),
        pl.BlockSpec((1, F_B, BLK), lambda i: (1, 0, i)),
        pl.BlockSpec((1, BLK), lambda i: (0, i)),
        pl.BlockSpec((F_B, BLK), lambda i: (0, i)),
        pl.BlockSpec((BLK, HEADS * BASES), lambda i: (i, 0)),
        pl.BlockSpec((BLK, 128), lambda i: (i, 0)),
        pl.BlockSpec((1, 128), lambda i: (0, 0)),
        pl.BlockSpec((1, 128), lambda i: (0, 0)),
        pl.BlockSpec((1, 128), lambda i: (0, 0)),
        pl.BlockSpec((BASES, HEADS * BASES, 128), lambda i: (0, 0, 0)),
        pl.BlockSpec((BASES, F_B, 128), lambda i: (0, 0, 0)),
    ],
    out_specs=pl.BlockSpec((BLK, 128), lambda i: (i, 0)),
    out_shape=jax.ShapeDtypeStruct((N, 128), jnp.float32),
)


def kernel(x, edge_index, W_bases, W_comb, b_comb, bias_conv, W_res, b_res,
           ln_gamma, ln_beta):
    ei_p = jnp.concatenate(
        [edge_index, jnp.full((2, EPAD - E), N, jnp.int32)], axis=1)
    row_p = ei_p[0].reshape(NCHUNKS, CHUNK)
    col_p = ei_p[1].reshape(NCHUNKS, CHUNK)

    bases_t, wt, res = _dense(x, W_bases, W_comb, b_comb.reshape(1, -1),
                              W_res, b_res.reshape(1, -1))
    degp = _sc_degree(col_p)
    b2_t, dis = _scale(degp, degp, bases_t)
    aggp = _sc_agg(b2_t, row_p, col_p)
    out = _finish(aggp, aggp, dis, bases_t, wt, res,
                  bias_conv.reshape(1, -1), ln_gamma.reshape(1, -1),
                  ln_beta.reshape(1, -1), jnp.asarray(_P_np), jnp.asarray(_Q_np))
    return out


# speedup vs baseline: 1.2349x; 1.2349x over previous
import functools

import jax
import jax.numpy as jnp
import numpy as np
from jax import lax
from jax.experimental import pallas as pl
from jax.experimental.pallas import tpu as pltpu
from jax.experimental.pallas import tpu_sc as plsc

N = 10000
NPAD = 10240
E = 320000
CHUNK = 128
NCHUNKS = 2560
CH_PER_TILE = NCHUNKS // 32
EPAD = NCHUNKS * CHUNK
HEADS = 8
BASES = 4
F_H = 16
F_B = BASES * F_H
ROWS_PER_TILE = NPAD // 16
BLK = 512
GRID = 20

_mesh = plsc.VectorSubcoreMesh(core_axis_name="c", subcore_axis_name="s")
_sc_params = pltpu.CompilerParams(use_tc_tiling_on_sc=False,
                                  needs_layout_passes=False)


@functools.partial(
    pl.kernel,
    out_type=jax.ShapeDtypeStruct((2, NPAD // 128, 128), jnp.float32),
    mesh=_mesh,
    compiler_params=_sc_params,
    scratch_types=[
        pltpu.VMEM((CH_PER_TILE, CHUNK), jnp.int32),
        pltpu.VMEM((CHUNK, 16), jnp.float32),
        pltpu.VMEM((ROWS_PER_TILE, 16), jnp.float32),
        pltpu.VMEM((ROWS_PER_TILE // 128, 128), jnp.float32),
        pltpu.VMEM_SHARED((NPAD, 16), jnp.float32),
        pltpu.SemaphoreType.DMA,
    ],
)
def _sc_degree(col_hbm, out_hbm, col_v, ones_v, deg_loc, compact_v, deg_sh, sem):
    cid = lax.axis_index("c")
    sid = lax.axis_index("s")
    wid = sid * 2 + cid

    z16 = jnp.zeros((16,), jnp.float32)

    @pl.loop(0, CHUNK)
    def _(i):
        ones_v[i, pl.ds(0, 16)] = z16

    @pl.loop(0, ROWS_PER_TILE // CHUNK)
    def _(k):
        pltpu.sync_copy(ones_v, deg_sh.at[pl.ds(sid * ROWS_PER_TILE + k * CHUNK, CHUNK)])

    o16 = jnp.ones((16,), jnp.float32)

    @pl.loop(0, CHUNK)
    def _(i):
        ones_v[i, pl.ds(0, 16)] = o16

    pltpu.sync_copy(col_hbm.at[pl.ds(wid * CH_PER_TILE, CH_PER_TILE)], col_v)
    plsc.subcore_barrier()

    @pl.loop(0, CH_PER_TILE)
    def _(j):
        pltpu.async_copy(ones_v, deg_sh.at[col_v.at[j]], sem, add=True)

    @pl.loop(0, CH_PER_TILE)
    def _(j):
        pltpu.make_async_copy(ones_v, deg_sh.at[col_v.at[j]], sem).wait()

    plsc.subcore_barrier()

    pltpu.sync_copy(deg_sh.at[pl.ds(sid * ROWS_PER_TILE, ROWS_PER_TILE)], deg_loc)
    ivec16 = lax.iota(jnp.int32, 16)
    zvec16 = jnp.zeros((16,), jnp.int32)

    @pl.loop(0, ROWS_PER_TILE // 128)
    def _(gg):
        for k in range(8):
            v = plsc.load_gather(deg_loc, [ivec16 + (gg * 128 + 16 * k), zvec16])
            compact_v[gg, pl.ds(k * 16, 16)] = v

    pltpu.sync_copy(
        compact_v,
        out_hbm.at[cid, pl.ds(sid * (ROWS_PER_TILE // 128), ROWS_PER_TILE // 128)],
    )


F_T = F_B // 16
ECH_ROWS = 16
ROWS_CORE = NCHUNKS // 2
NECH = ROWS_CORE // ECH_ROWS


@functools.partial(
    pl.kernel,
    out_type=jax.ShapeDtypeStruct((2, F_B, NPAD), jnp.float32),
    mesh=_mesh,
    compiler_params=_sc_params,
    scratch_types=[
        pltpu.VMEM((F_T, NPAD), jnp.float32),
        pltpu.VMEM((F_T, NPAD), jnp.float32),
        pltpu.VMEM((2, ECH_ROWS, CHUNK), jnp.int32),
        pltpu.VMEM((2, ECH_ROWS, CHUNK), jnp.int32),
        pltpu.SemaphoreType.DMA,
        pltpu.SemaphoreType.DMA,
        pltpu.SemaphoreType.DMA,
    ],
)
def _sc_agg(b2_hbm, row_hbm, col_hbm, out_hbm, b2_v, agg_v, row_v, col_v,
            sem_r, sem_c, sem_b):
    cid = lax.axis_index("c")
    sid = lax.axis_index("s")

    b2cp = pltpu.async_copy(b2_hbm.at[pl.ds(F_T * sid, F_T)], b2_v, sem_b)

    z16 = jnp.zeros((16,), jnp.float32)
    for j in range(F_T):
        @pl.loop(0, NPAD, step=16)
        def _(i):
            agg_v[j, pl.ds(i, 16)] = z16

    rbase = cid * ROWS_CORE

    def _fetch(ch, slot):
        pltpu.async_copy(
            row_hbm.at[pl.ds(rbase + ch * ECH_ROWS, ECH_ROWS)], row_v.at[slot], sem_r)
        pltpu.async_copy(
            col_hbm.at[pl.ds(rbase + ch * ECH_ROWS, ECH_ROWS)], col_v.at[slot], sem_c)

    def _wait(ch, slot):
        pltpu.make_async_copy(
            row_hbm.at[pl.ds(rbase + ch * ECH_ROWS, ECH_ROWS)], row_v.at[slot], sem_r).wait()
        pltpu.make_async_copy(
            col_hbm.at[pl.ds(rbase + ch * ECH_ROWS, ECH_ROWS)], col_v.at[slot], sem_c).wait()

    jvecs = [jnp.full((16,), j, jnp.int32) for j in range(F_T)]

    def _process(slot):
        @plsc.parallel_loop(0, ECH_ROWS, unroll=4)
        def _(r):
            for k in range(CHUNK // 16):
                r16 = row_v[slot, r, pl.ds(k * 16, 16)]
                c16 = col_v[slot, r, pl.ds(k * 16, 16)]
                for j in range(F_T):
                    v = plsc.load_gather(b2_v, [jvecs[j], r16])
                    plsc.addupdate_scatter(agg_v, [jvecs[j], c16], v)

    _fetch(0, 0)
    b2cp.wait()

    @pl.loop(0, NECH, step=2)
    def _(ch):
        _wait(ch, 0)
        _fetch(ch + 1, 1)
        _process(0)
        _wait(ch + 1, 1)

        @pl.when(ch + 2 < NECH)
        def _():
            _fetch(ch + 2, 0)

        _process(1)

    pltpu.sync_copy(agg_v, out_hbm.at[cid, pl.ds(F_T * sid, F_T)])


def _dense_body(x_ref, wb_ref, wc_ref, bc_ref, wr_ref, br_ref, bt_ref, wt_ref, r_ref):
    xb = x_ref[...]
    bt_ref[...] = lax.dot_general(wb_ref[...], xb, (((0,), (1,)), ((), ())),
                                  preferred_element_type=jnp.float32)
    wt_ref[...] = jnp.dot(xb, wc_ref[...], preferred_element_type=jnp.float32) + bc_ref[...]
    r_ref[...] = jnp.dot(xb, wr_ref[...], preferred_element_type=jnp.float32) + br_ref[...]


_dense = pl.pallas_call(
    _dense_body,
    grid=(GRID,),
    in_specs=[
        pl.BlockSpec((BLK, 128), lambda i: (i, 0)),
        pl.BlockSpec((128, F_B), lambda i: (0, 0)),
        pl.BlockSpec((128, HEADS * BASES), lambda i: (0, 0)),
        pl.BlockSpec((1, HEADS * BASES), lambda i: (0, 0)),
        pl.BlockSpec((128, 128), lambda i: (0, 0)),
        pl.BlockSpec((1, 128), lambda i: (0, 0)),
    ],
    out_specs=[
        pl.BlockSpec((F_B, BLK), lambda i: (0, i)),
        pl.BlockSpec((BLK, HEADS * BASES), lambda i: (i, 0)),
        pl.BlockSpec((BLK, 128), lambda i: (i, 0)),
    ],
    out_shape=[
        jax.ShapeDtypeStruct((F_B, NPAD), jnp.float32),
        jax.ShapeDtypeStruct((N, HEADS * BASES), jnp.float32),
        jax.ShapeDtypeStruct((N, 128), jnp.float32),
    ],
)


SBLK = 1024


def _scale_body(dp0_ref, dp1_ref, bases_ref, b2_ref, dis_ref):
    d = dp0_ref[...][0] + dp1_ref[...][0]
    deg_row = jnp.concatenate([d[r:r + 1, :] for r in range(8)], axis=1) + 1.0
    dis_row = lax.rsqrt(deg_row)
    dis_ref[...] = dis_row
    b2_ref[...] = bases_ref[...] * dis_row


_scale = pl.pallas_call(
    _scale_body,
    grid=(NPAD // SBLK,),
    in_specs=[
        pl.BlockSpec((1, 8, 128), lambda i: (0, i, 0)),
        pl.BlockSpec((1, 8, 128), lambda i: (1, i, 0)),
        pl.BlockSpec((F_B, SBLK), lambda i: (0, i)),
    ],
    out_specs=[
        pl.BlockSpec((F_B, SBLK), lambda i: (0, i)),
        pl.BlockSpec((1, SBLK), lambda i: (0, i)),
    ],
    out_shape=[
        jax.ShapeDtypeStruct((F_B, NPAD), jnp.float32),
        jax.ShapeDtypeStruct((1, NPAD), jnp.float32),
    ],
)


_P_np = np.zeros((BASES, HEADS * BASES, 128), np.float32)
_Q_np = np.zeros((BASES, F_B, 128), np.float32)
for _b in range(BASES):
    for _h in range(HEADS):
        for _f in range(F_H):
            _P_np[_b, _h * BASES + _b, _h * F_H + _f] = 1.0
            _Q_np[_b, _b * F_H + _f, _h * F_H + _f] = 1.0


def _finish_body(a0_ref, a1_ref, dis_ref, bases_ref, wt_ref, res_ref, bc_ref,
                 g_ref, bt_ref, p_ref, q_ref, o_ref):
    disr = dis_ref[...]
    a_t = a0_ref[...][0] + a1_ref[...][0]
    aggf_t = disr * a_t + (disr * disr) * bases_ref[...]
    wt = wt_ref[...]
    conv = None
    for b in range(BASES):
        we = jnp.dot(wt, p_ref[b], preferred_element_type=jnp.float32)
        ae = lax.dot_general(aggf_t, q_ref[b], (((0,), (0,)), ((), ())),
                             preferred_element_type=jnp.float32)
        t = we * ae
        conv = t if conv is None else conv + t
    o = conv + bc_ref[...] + res_ref[...]
    mu = jnp.mean(o, axis=1, keepdims=True)
    var = jnp.mean((o - mu) * (o - mu), axis=1, keepdims=True)
    o = (o - mu) * lax.rsqrt(var + 1e-5) * g_ref[...] + bt_ref[...]
    o_ref[...] = jnp.maximum(o, 0.0)


_finish = pl.pallas_call(
    _finish_body,
    grid=(GRID,),
    in_specs=[
        pl.BlockSpec((1, F_B, BLK), lambda i: (0, 0, i)),
        pl.BlockSpec((1, F_B, BLK), lambda i: (1, 0, i)),
        pl.BlockSpec((1, BLK), lambda i: (0, i)),
        pl.BlockSpec((F_B, BLK), lambda i: (0, i)),
        pl.BlockSpec((BLK, HEADS * BASES), lambda i: (i, 0)),
        pl.BlockSpec((BLK, 128), lambda i: (i, 0)),
        pl.BlockSpec((1, 128), lambda i: (0, 0)),
        pl.BlockSpec((1, 128), lambda i: (0, 0)),
        pl.BlockSpec((1, 128), lambda i: (0, 0)),
        pl.BlockSpec((BASES, HEADS * BASES, 128), lambda i: (0, 0, 0)),
        pl.BlockSpec((BASES, F_B, 128), lambda i: (0, 0, 0)),
    ],
    out_specs=pl.BlockSpec((BLK, 128), lambda i: (i, 0)),
    out_shape=jax.ShapeDtypeStruct((N, 128), jnp.float32),
)


def kernel(x, edge_index, W_bases, W_comb, b_comb, bias_conv, W_res, b_res,
           ln_gamma, ln_beta):
    ei_p = jnp.concatenate(
        [edge_index, jnp.full((2, EPAD - E), N, jnp.int32)], axis=1)
    row_p = ei_p[0].reshape(NCHUNKS, CHUNK)
    col_p = ei_p[1].reshape(NCHUNKS, CHUNK)

    bases_t, wt, res = _dense(x, W_bases, W_comb, b_comb.reshape(1, -1),
                              W_res, b_res.reshape(1, -1))
    degp = _sc_degree(col_p)
    b2_t, dis = _scale(degp, degp, bases_t)
    aggp = _sc_agg(b2_t, row_p, col_p)
    out = _finish(aggp, aggp, dis, bases_t, wt, res,
                  bias_conv.reshape(1, -1), ln_gamma.reshape(1, -1),
                  ln_beta.reshape(1, -1), jnp.asarray(_P_np), jnp.asarray(_Q_np))
    return out
